# baseline (device time: 147510 ns/iter reference)
import jax
import jax.numpy as jnp
from jax import lax
from jax.experimental import pallas as pl
from jax.experimental.pallas import tpu as pltpu

N_DEV = 32
HEADS_PER = 4
DH = 64
HD = HEADS_PER * DH
WINDOW = 128
CHUNK_R = 64
CHUNK_C = 128


def _allreduce_body(p_ref, out_ref, acc_ref, rs_recv_ref,
                    rs_send_sems, rs_recv_sems, ag_send_sems, ag_recv_sems):
    my = lax.axis_index("i")
    left = (my - 1) % N_DEV
    right = (my + 1) % N_DEV

    barrier = pltpu.get_barrier_semaphore()
    for nbr in (left, right):
        pl.semaphore_signal(barrier, inc=1, device_id=(nbr,),
                            device_id_type=pl.DeviceIdType.MESH)
    pl.semaphore_wait(barrier, 2)

    acc_ref[...] = p_ref[...]

    for s in range(N_DEV - 1):
        send_idx = (my - s) % N_DEV
        recv_idx = (my - s - 1) % N_DEV
        rdma = pltpu.make_async_remote_copy(
            src_ref=acc_ref.at[send_idx],
            dst_ref=rs_recv_ref.at[s],
            send_sem=rs_send_sems.at[s],
            recv_sem=rs_recv_sems.at[s],
            device_id=(right,),
            device_id_type=pl.DeviceIdType.MESH,
        )
        rdma.start()
        rdma.wait()
        acc_ref[recv_idx] = acc_ref[recv_idx] + rs_recv_ref[s]

    own = (my + 1) % N_DEV
    out_ref[own] = acc_ref[own]

    for s in range(N_DEV - 1):
        g = (my + 1 - s) % N_DEV
        rdma = pltpu.make_async_remote_copy(
            src_ref=out_ref.at[g],
            dst_ref=out_ref.at[g],
            send_sem=ag_send_sems.at[s],
            recv_sem=ag_recv_sems.at[s],
            device_id=(right,),
            device_id_type=pl.DeviceIdType.MESH,
        )
        rdma.start()
        rdma.wait()


def kernel(x, Wq, K_ext, V_ext, Wo):
    my = lax.axis_index("i")
    B, Sq, D = x.shape
    Skv = K_ext.shape[1]

    xb = x.astype(jnp.bfloat16)
    Wq_s = lax.dynamic_slice(Wq, (0, my * HD), (D, HD)).astype(jnp.bfloat16)
    Q = jnp.einsum("bsd,dh->bsh", xb, Wq_s,
                   preferred_element_type=jnp.float32)
    Q = Q.reshape(B, Sq, HEADS_PER, DH).astype(jnp.bfloat16)
    K = K_ext.astype(jnp.bfloat16)
    V = V_ext.astype(jnp.bfloat16)

    scores = jnp.einsum("bihd,bjhd->bhij", Q, K,
                        preferred_element_type=jnp.float32) * 0.125
    qi = lax.broadcasted_iota(jnp.int32, (Sq, Skv), 0)
    ki = lax.broadcasted_iota(jnp.int32, (Sq, Skv), 1)
    mask = jnp.abs(qi - ki) <= WINDOW
    scores = jnp.where(mask[None, None, :, :], scores, -1e9)
    w = jax.nn.softmax(scores, axis=-1)

    ctx = jnp.einsum("bhij,bjhd->bihd", w.astype(jnp.bfloat16), V,
                     preferred_element_type=jnp.float32)
    ctx = ctx.reshape(B, Sq, HD).astype(jnp.bfloat16)
    Wo_s = lax.dynamic_slice(Wo, (my * HD, 0), (HD, D)).astype(jnp.bfloat16)
    partial = jnp.einsum("bsh,hd->bsd", ctx, Wo_s,
                         preferred_element_type=jnp.float32)

    p = partial.reshape(N_DEV, CHUNK_R, CHUNK_C)

    out = pl.pallas_call(
        _allreduce_body,
        out_shape=jax.ShapeDtypeStruct((N_DEV, CHUNK_R, CHUNK_C), jnp.float32),
        in_specs=[pl.BlockSpec(memory_space=pltpu.VMEM)],
        out_specs=pl.BlockSpec(memory_space=pltpu.VMEM),
        scratch_shapes=[
            pltpu.VMEM((N_DEV, CHUNK_R, CHUNK_C), jnp.float32),
            pltpu.VMEM((N_DEV - 1, CHUNK_R, CHUNK_C), jnp.float32),
            pltpu.SemaphoreType.DMA((N_DEV - 1,)),
            pltpu.SemaphoreType.DMA((N_DEV - 1,)),
            pltpu.SemaphoreType.DMA((N_DEV - 1,)),
            pltpu.SemaphoreType.DMA((N_DEV - 1,)),
        ],
        compiler_params=pltpu.CompilerParams(collective_id=0),
    )(p)
    return out.reshape(B, Sq, D)


# device time: 53058 ns/iter; 2.7802x vs baseline; 2.7802x over previous
import jax
import jax.numpy as jnp
from jax import lax
from jax.experimental import pallas as pl
from jax.experimental.pallas import tpu as pltpu

N_DEV = 32
HEADS_PER = 4
DH = 64
HD = HEADS_PER * DH
WINDOW = 128
CHUNK_R = 64
CHUNK_C = 128


LOG_N = 5


def _allreduce_body(p_ref, out_ref, acc_ref, rs_recv_ref,
                    rs_send_sems, rs_recv_sems, ag_send_sems, ag_recv_sems):
    my = lax.axis_index("i")

    barrier = pltpu.get_barrier_semaphore()
    for k in range(LOG_N):
        pl.semaphore_signal(barrier, inc=1, device_id=(my ^ (1 << k),),
                            device_id_type=pl.DeviceIdType.MESH)
    pl.semaphore_wait(barrier, LOG_N)

    acc_ref[...] = p_ref[...]

    base = 0
    recv_off = 0
    for k in range(LOG_N):
        m = (N_DEV // 2) >> k
        bit = (my >> k) & 1
        send_start = base + (1 - bit) * m
        keep_start = base + bit * m
        rdma = pltpu.make_async_remote_copy(
            src_ref=acc_ref.at[pl.ds(send_start, m)],
            dst_ref=rs_recv_ref.at[pl.ds(recv_off, m)],
            send_sem=rs_send_sems.at[k],
            recv_sem=rs_recv_sems.at[k],
            device_id=(my ^ (1 << k),),
            device_id_type=pl.DeviceIdType.MESH,
        )
        rdma.start()
        rdma.wait()
        acc_ref[pl.ds(keep_start, m)] = (
            acc_ref[pl.ds(keep_start, m)] + rs_recv_ref[pl.ds(recv_off, m)]
        )
        base = keep_start
        recv_off += m

    out_ref[pl.ds(base, 1)] = acc_ref[pl.ds(base, 1)]

    own_base = base
    for k in reversed(range(LOG_N)):
        m = (N_DEV // 2) >> k
        rdma = pltpu.make_async_remote_copy(
            src_ref=out_ref.at[pl.ds(own_base, m)],
            dst_ref=out_ref.at[pl.ds(own_base, m)],
            send_sem=ag_send_sems.at[k],
            recv_sem=ag_recv_sems.at[k],
            device_id=(my ^ (1 << k),),
            device_id_type=pl.DeviceIdType.MESH,
        )
        rdma.start()
        rdma.wait()
        own_base = own_base - ((my >> k) & 1) * m


def kernel(x, Wq, K_ext, V_ext, Wo):
    my = lax.axis_index("i")
    B, Sq, D = x.shape
    Skv = K_ext.shape[1]

    xb = x.astype(jnp.bfloat16)
    Wq_s = lax.dynamic_slice(Wq, (0, my * HD), (D, HD)).astype(jnp.bfloat16)
    Q = jnp.einsum("bsd,dh->bsh", xb, Wq_s,
                   preferred_element_type=jnp.float32)
    Q = Q.reshape(B, Sq, HEADS_PER, DH).astype(jnp.bfloat16)
    K = K_ext.astype(jnp.bfloat16)
    V = V_ext.astype(jnp.bfloat16)

    scores = jnp.einsum("bihd,bjhd->bhij", Q, K,
                        preferred_element_type=jnp.float32) * 0.125
    qi = lax.broadcasted_iota(jnp.int32, (Sq, Skv), 0)
    ki = lax.broadcasted_iota(jnp.int32, (Sq, Skv), 1)
    mask = jnp.abs(qi - ki) <= WINDOW
    scores = jnp.where(mask[None, None, :, :], scores, -1e9)
    w = jax.nn.softmax(scores, axis=-1)

    ctx = jnp.einsum("bhij,bjhd->bihd", w.astype(jnp.bfloat16), V,
                     preferred_element_type=jnp.float32)
    ctx = ctx.reshape(B, Sq, HD).astype(jnp.bfloat16)
    Wo_s = lax.dynamic_slice(Wo, (my * HD, 0), (HD, D)).astype(jnp.bfloat16)
    partial = jnp.einsum("bsh,hd->bsd", ctx, Wo_s,
                         preferred_element_type=jnp.float32)

    p = partial.reshape(N_DEV, CHUNK_R, CHUNK_C)

    out = pl.pallas_call(
        _allreduce_body,
        out_shape=jax.ShapeDtypeStruct((N_DEV, CHUNK_R, CHUNK_C), jnp.float32),
        in_specs=[pl.BlockSpec(memory_space=pltpu.VMEM)],
        out_specs=pl.BlockSpec(memory_space=pltpu.VMEM),
        scratch_shapes=[
            pltpu.VMEM((N_DEV, CHUNK_R, CHUNK_C), jnp.float32),
            pltpu.VMEM((N_DEV - 1, CHUNK_R, CHUNK_C), jnp.float32),
            pltpu.SemaphoreType.DMA((LOG_N,)),
            pltpu.SemaphoreType.DMA((LOG_N,)),
            pltpu.SemaphoreType.DMA((LOG_N,)),
            pltpu.SemaphoreType.DMA((LOG_N,)),
        ],
        compiler_params=pltpu.CompilerParams(collective_id=0),
    )(p)
    return out.reshape(B, Sq, D)


# device time: 44705 ns/iter; 3.2996x vs baseline; 1.1868x over previous
import jax
import jax.numpy as jnp
from jax import lax
from jax.experimental import pallas as pl
from jax.experimental.pallas import tpu as pltpu

N_DEV = 32
HEADS_PER = 4
DH = 64
HD = HEADS_PER * DH
WINDOW = 128
CHUNK_R = 64
CHUNK_C = 128


LOG_N = 5


def _allreduce_body(p_ref, out_ref, acc_ref, rs_recv_ref,
                    bf_send_ref, bf_recv_ref,
                    rs_send_sems, rs_recv_sems, ag_send_sems, ag_recv_sems):
    my = lax.axis_index("i")

    barrier = pltpu.get_barrier_semaphore()
    for k in range(LOG_N):
        pl.semaphore_signal(barrier, inc=1, device_id=(my ^ (1 << k),),
                            device_id_type=pl.DeviceIdType.MESH)
    pl.semaphore_wait(barrier, LOG_N)

    acc_ref[...] = p_ref[...]

    base = 0
    recv_off = 0
    for k in range(LOG_N):
        m = (N_DEV // 2) >> k
        bit = (my >> k) & 1
        send_start = base + (1 - bit) * m
        keep_start = base + bit * m
        if k == 0:
            bf_send_ref[...] = acc_ref[pl.ds(send_start, m)].astype(
                jnp.bfloat16)
            src, dst = bf_send_ref, bf_recv_ref
        else:
            src = acc_ref.at[pl.ds(send_start, m)]
            dst = rs_recv_ref.at[pl.ds(recv_off, m)]
        rdma = pltpu.make_async_remote_copy(
            src_ref=src,
            dst_ref=dst,
            send_sem=rs_send_sems.at[k],
            recv_sem=rs_recv_sems.at[k],
            device_id=(my ^ (1 << k),),
            device_id_type=pl.DeviceIdType.MESH,
        )
        rdma.start()
        rdma.wait()
        if k == 0:
            add = bf_recv_ref[...].astype(jnp.float32)
        else:
            add = rs_recv_ref[pl.ds(recv_off, m)]
            recv_off += m
        acc_ref[pl.ds(keep_start, m)] = acc_ref[pl.ds(keep_start, m)] + add
        base = keep_start

    out_ref[pl.ds(base, 1)] = acc_ref[pl.ds(base, 1)].astype(jnp.bfloat16)

    own_base = base
    for k in reversed(range(LOG_N)):
        m = (N_DEV // 2) >> k
        rdma = pltpu.make_async_remote_copy(
            src_ref=out_ref.at[pl.ds(own_base, m)],
            dst_ref=out_ref.at[pl.ds(own_base, m)],
            send_sem=ag_send_sems.at[k],
            recv_sem=ag_recv_sems.at[k],
            device_id=(my ^ (1 << k),),
            device_id_type=pl.DeviceIdType.MESH,
        )
        rdma.start()
        rdma.wait()
        own_base = own_base - ((my >> k) & 1) * m


def kernel(x, Wq, K_ext, V_ext, Wo):
    my = lax.axis_index("i")
    B, Sq, D = x.shape
    Skv = K_ext.shape[1]

    xb = x.astype(jnp.bfloat16)
    Wq_s = lax.dynamic_slice(Wq, (0, my * HD), (D, HD)).astype(jnp.bfloat16)
    Q = jnp.einsum("bsd,dh->bsh", xb, Wq_s,
                   preferred_element_type=jnp.float32)
    Q = Q.reshape(B, Sq, HEADS_PER, DH).astype(jnp.bfloat16)
    K = K_ext.astype(jnp.bfloat16)
    V = V_ext.astype(jnp.bfloat16)

    scores = jnp.einsum("bihd,bjhd->bhij", Q, K,
                        preferred_element_type=jnp.float32) * 0.125
    qi = lax.broadcasted_iota(jnp.int32, (Sq, Skv), 0)
    ki = lax.broadcasted_iota(jnp.int32, (Sq, Skv), 1)
    mask = jnp.abs(qi - ki) <= WINDOW
    scores = jnp.where(mask[None, None, :, :], scores, -1e9)
    w = jax.nn.softmax(scores, axis=-1)

    ctx = jnp.einsum("bhij,bjhd->bihd", w.astype(jnp.bfloat16), V,
                     preferred_element_type=jnp.float32)
    ctx = ctx.reshape(B, Sq, HD).astype(jnp.bfloat16)
    Wo_s = lax.dynamic_slice(Wo, (my * HD, 0), (HD, D)).astype(jnp.bfloat16)
    partial = jnp.einsum("bsh,hd->bsd", ctx, Wo_s,
                         preferred_element_type=jnp.float32)

    p = partial.reshape(N_DEV, CHUNK_R, CHUNK_C)

    out = pl.pallas_call(
        _allreduce_body,
        out_shape=jax.ShapeDtypeStruct((N_DEV, CHUNK_R, CHUNK_C),
                                       jnp.bfloat16),
        in_specs=[pl.BlockSpec(memory_space=pltpu.VMEM)],
        out_specs=pl.BlockSpec(memory_space=pltpu.VMEM),
        scratch_shapes=[
            pltpu.VMEM((N_DEV, CHUNK_R, CHUNK_C), jnp.float32),
            pltpu.VMEM((N_DEV // 2 - 1, CHUNK_R, CHUNK_C), jnp.float32),
            pltpu.VMEM((N_DEV // 2, CHUNK_R, CHUNK_C), jnp.bfloat16),
            pltpu.VMEM((N_DEV // 2, CHUNK_R, CHUNK_C), jnp.bfloat16),
            pltpu.SemaphoreType.DMA((LOG_N,)),
            pltpu.SemaphoreType.DMA((LOG_N,)),
            pltpu.SemaphoreType.DMA((LOG_N,)),
            pltpu.SemaphoreType.DMA((LOG_N,)),
        ],
        compiler_params=pltpu.CompilerParams(collective_id=0),
    )(p)
    return out.reshape(B, Sq, D).astype(jnp.float32)


# device time: 30858 ns/iter; 4.7803x vs baseline; 1.4487x over previous
import jax
import jax.numpy as jnp
from jax import lax
from jax.experimental import pallas as pl
from jax.experimental.pallas import tpu as pltpu

N_DEV = 32
HEADS_PER = 4
DH = 64
HD = HEADS_PER * DH
WINDOW = 128
CHUNK_R = 64
CHUNK_C = 128


def _allreduce_body(p_ref, out_ref, pbf_ref, rs_recv_ref,
                    rs_send_sems, rs_recv_sems, ag_send_sems, ag_recv_sems):
    my = lax.axis_index("i")

    barrier = pltpu.get_barrier_semaphore()
    for s in range(1, N_DEV):
        pl.semaphore_signal(barrier, inc=1, device_id=((my + s) % N_DEV,),
                            device_id_type=pl.DeviceIdType.MESH)
    pl.semaphore_wait(barrier, N_DEV - 1)

    pbf_ref[...] = p_ref[...].astype(jnp.bfloat16)

    rs = []
    for s in range(1, N_DEV):
        peer = (my + s) % N_DEV
        rdma = pltpu.make_async_remote_copy(
            src_ref=pbf_ref.at[pl.ds(peer, 1)],
            dst_ref=rs_recv_ref.at[pl.ds(s, 1)],
            send_sem=rs_send_sems.at[s],
            recv_sem=rs_recv_sems.at[s],
            device_id=(peer,),
            device_id_type=pl.DeviceIdType.MESH,
        )
        rdma.start()
        rs.append(rdma)
    for rdma in rs:
        rdma.wait()

    reduced = p_ref[pl.ds(my, 1)] + jnp.sum(
        rs_recv_ref[pl.ds(1, N_DEV - 1)].astype(jnp.float32),
        axis=0, keepdims=True)
    out_ref[pl.ds(my, 1)] = reduced.astype(jnp.bfloat16)

    ag = []
    for s in range(1, N_DEV):
        peer = (my + s) % N_DEV
        rdma = pltpu.make_async_remote_copy(
            src_ref=out_ref.at[pl.ds(my, 1)],
            dst_ref=out_ref.at[pl.ds(my, 1)],
            send_sem=ag_send_sems.at[s],
            recv_sem=ag_recv_sems.at[s],
            device_id=(peer,),
            device_id_type=pl.DeviceIdType.MESH,
        )
        rdma.start()
        ag.append(rdma)
    for rdma in ag:
        rdma.wait()


def kernel(x, Wq, K_ext, V_ext, Wo):
    my = lax.axis_index("i")
    B, Sq, D = x.shape
    Skv = K_ext.shape[1]

    xb = x.astype(jnp.bfloat16)
    Wq_s = lax.dynamic_slice(Wq, (0, my * HD), (D, HD)).astype(jnp.bfloat16)
    Q = jnp.einsum("bsd,dh->bsh", xb, Wq_s,
                   preferred_element_type=jnp.float32)
    Q = Q.reshape(B, Sq, HEADS_PER, DH).astype(jnp.bfloat16)
    K = K_ext.astype(jnp.bfloat16)
    V = V_ext.astype(jnp.bfloat16)

    scores = jnp.einsum("bihd,bjhd->bhij", Q, K,
                        preferred_element_type=jnp.float32) * 0.125
    qi = lax.broadcasted_iota(jnp.int32, (Sq, Skv), 0)
    ki = lax.broadcasted_iota(jnp.int32, (Sq, Skv), 1)
    mask = jnp.abs(qi - ki) <= WINDOW
    scores = jnp.where(mask[None, None, :, :], scores, -1e9)
    w = jax.nn.softmax(scores, axis=-1)

    ctx = jnp.einsum("bhij,bjhd->bihd", w.astype(jnp.bfloat16), V,
                     preferred_element_type=jnp.float32)
    ctx = ctx.reshape(B, Sq, HD).astype(jnp.bfloat16)
    Wo_s = lax.dynamic_slice(Wo, (my * HD, 0), (HD, D)).astype(jnp.bfloat16)
    partial = jnp.einsum("bsh,hd->bsd", ctx, Wo_s,
                         preferred_element_type=jnp.float32)

    p = partial.reshape(N_DEV, CHUNK_R, CHUNK_C)

    out = pl.pallas_call(
        _allreduce_body,
        out_shape=jax.ShapeDtypeStruct((N_DEV, CHUNK_R, CHUNK_C),
                                       jnp.bfloat16),
        in_specs=[pl.BlockSpec(memory_space=pltpu.VMEM)],
        out_specs=pl.BlockSpec(memory_space=pltpu.VMEM),
        scratch_shapes=[
            pltpu.VMEM((N_DEV, CHUNK_R, CHUNK_C), jnp.bfloat16),
            pltpu.VMEM((N_DEV, CHUNK_R, CHUNK_C), jnp.bfloat16),
            pltpu.SemaphoreType.DMA((N_DEV,)),
            pltpu.SemaphoreType.DMA((N_DEV,)),
            pltpu.SemaphoreType.DMA((N_DEV,)),
            pltpu.SemaphoreType.DMA((N_DEV,)),
        ],
        compiler_params=pltpu.CompilerParams(collective_id=0),
    )(p)
    return out.reshape(B, Sq, D).astype(jnp.float32)
